# Initial kernel scaffold; baseline (speedup 1.0000x reference)
import jax
import jax.numpy as jnp
from jax.experimental import pallas as pl


def kernel(queries, bank_features, bank_probs):
    raise NotImplementedError("write your pallas kernel here")


# speedup vs baseline: 1.0607x; 1.0607x over previous
import functools

import jax
import jax.numpy as jnp
from jax import lax
from jax.experimental import pallas as pl
from jax.experimental.pallas import tpu as pltpu
from jax.experimental.pallas import tpu_sc as plsc

Q = 1024
K = 100000
D = 128
C = 1000
NN = 8

BK = 2048
NB = (K + BK - 1) // BK

NEG = -3.0e38
BIGI = 2**30


def _topk_body(qn_ref, kb_ref, idx_ref, cand_d, cand_i):
    b = pl.program_id(0)
    kb = kb_ref[...]
    qn = qn_ref[...]
    norm = jnp.sqrt(jnp.sum(kb * kb, axis=1, keepdims=True))
    knb = kb / jnp.maximum(norm, 1e-12)
    sim = lax.dot_general(qn, knb, (((1,), (1,)), ((), ())),
                          preferred_element_type=jnp.float32)
    d = 1.0 - sim

    gidx = b * BK + lax.broadcasted_iota(jnp.int32, (Q, BK), 1)
    d = jnp.where(gidx < K, d, NEG)

    for t in range(NN):
        m = jnp.max(d, axis=1, keepdims=True)
        sel = jnp.min(jnp.where(d == m, gidx, BIGI), axis=1,
                      keepdims=True)
        cand_d[:, NN + t : NN + t + 1] = m
        cand_i[:, NN + t : NN + t + 1] = sel
        d = jnp.where(gidx == sel, NEG, d)

    @pl.when(b == 0)
    def _init():
        cand_d[:, 0:NN] = jnp.full((Q, NN), NEG, jnp.float32)
        cand_i[:, 0:NN] = jnp.full((Q, NN), BIGI, jnp.int32)

    cd = cand_d[...]
    ci = cand_i[...]
    for t in range(NN):
        m = jnp.max(cd, axis=1, keepdims=True)
        sel = jnp.min(jnp.where(cd == m, ci, BIGI), axis=1, keepdims=True)
        cand_d[:, t : t + 1] = m
        cand_i[:, t : t + 1] = sel
        cd = jnp.where(ci == sel, NEG, cd)

    @pl.when(b == NB - 1)
    def _emit():
        idx_ref[...] = cand_i[:, 0:NN]


def _topk_indices(qn, bank_features):
    return pl.pallas_call(
        _topk_body,
        grid=(NB,),
        in_specs=[
            pl.BlockSpec((Q, D), lambda b: (0, 0)),
            pl.BlockSpec((BK, D), lambda b: (b, 0)),
        ],
        out_specs=pl.BlockSpec((Q, NN), lambda b: (0, 0)),
        out_shape=jax.ShapeDtypeStruct((Q, NN), jnp.int32),
        scratch_shapes=[
            pltpu.VMEM((Q, 2 * NN), jnp.float32),
            pltpu.VMEM((Q, 2 * NN), jnp.int32),
        ],
    )(qn, bank_features)


_NC, _NS, _L = 2, 16, 16
_NW = _NC * _NS
_QPW = Q // _NW

_C_OFFS = [o * _L for o in range(C // _L)] + ([C - _L] if C % _L else [])


def _gather_body(idx_hbm, probs_hbm, feats_hbm,
                 lbl_out, probs_out, feats_out,
                 idx_v, prow_v, frow_v, pout_v, fout_v, lbl_v,
                 semp, semf):
    wid = lax.axis_index("s") * _NC + lax.axis_index("c")
    base = wid * _QPW
    pltpu.sync_copy(idx_hbm.at[pl.ds(base, _QPW)], idx_v)

    lane = lax.iota(jnp.int32, _L)
    inv = jnp.float32(1.0 / NN)

    def one_query(ql, _):
        idxrow = idx_v.at[ql]
        cp = pltpu.async_copy(probs_hbm.at[idxrow], prow_v, semp)
        cf = pltpu.async_copy(feats_hbm.at[idxrow], frow_v, semf)
        cp.wait()
        cf.wait()

        maxv = jnp.full((_L,), NEG, jnp.float32)
        maxi = jnp.zeros((_L,), jnp.int32)
        for off in _C_OFFS:
            s = jnp.zeros((_L,), jnp.float32)
            for r in range(NN):
                s = s + prow_v[r, pl.ds(off, _L)]
            s = s * inv
            pout_v[pl.ds(off, _L)] = s
            idx16 = lane + off
            better = s > maxv
            maxv = jnp.where(better, s, maxv)
            maxi = jnp.where(better, idx16, maxi)
        def _allmax(v):
            return plsc.cummax(lax.rev(plsc.cummax(v), dimensions=(0,)))

        m_vec = _allmax(maxv)
        cand = jnp.where(maxv == m_vec, maxi, BIGI)
        lab_vec = -_allmax(-cand)
        plsc.store_scatter(lbl_v,
                           [jnp.full((_L,), ql, jnp.int32)],
                           lab_vec,
                           mask=lane == 0)

        for off in range(0, D, _L):
            s = jnp.zeros((_L,), jnp.float32)
            for r in range(NN):
                s = s + frow_v[r, pl.ds(off, _L)]
            fout_v[pl.ds(off, _L)] = s * inv

        qg = base + ql
        pltpu.sync_copy(pout_v, probs_out.at[qg])
        pltpu.sync_copy(fout_v, feats_out.at[qg])
        return ()

    lax.fori_loop(0, _QPW, one_query, (), unroll=False)
    pltpu.sync_copy(lbl_v, lbl_out.at[pl.ds(base, _QPW)])


@functools.lru_cache(maxsize=1)
def _gather_mean_kernel():
    return pl.kernel(
        _gather_body,
        out_type=(
            jax.ShapeDtypeStruct((Q,), jnp.int32),
            jax.ShapeDtypeStruct((Q, C), jnp.float32),
            jax.ShapeDtypeStruct((Q, D), jnp.float32),
        ),
        mesh=plsc.VectorSubcoreMesh(core_axis_name="c",
                                    subcore_axis_name="s"),
        compiler_params=pltpu.CompilerParams(needs_layout_passes=False,
                                             use_tc_tiling_on_sc=False),
        scratch_types=[
            pltpu.VMEM((_QPW, NN), jnp.int32),
            pltpu.VMEM((NN, C), jnp.float32),
            pltpu.VMEM((NN, D), jnp.float32),
            pltpu.VMEM((C,), jnp.float32),
            pltpu.VMEM((D,), jnp.float32),
            pltpu.VMEM((_QPW,), jnp.int32),
            pltpu.SemaphoreType.DMA,
            pltpu.SemaphoreType.DMA,
        ],
    )


def kernel(queries, bank_features, bank_probs):
    qnorm = jnp.linalg.norm(queries, axis=1, keepdims=True)
    qn = queries / jnp.maximum(qnorm, 1e-12)
    idxs = _topk_indices(qn, bank_features)
    pred_labels, pred_probs, grads = _gather_mean_kernel()(
        idxs, bank_probs, bank_features)
    return (pred_labels, pred_probs, grads)


# speedup vs baseline: 1.3418x; 1.2650x over previous
import functools

import jax
import jax.numpy as jnp
from jax import lax
from jax.experimental import pallas as pl
from jax.experimental.pallas import tpu as pltpu
from jax.experimental.pallas import tpu_sc as plsc

Q = 1024
K = 100000
D = 128
C = 1000
NN = 8

BK = 2048
NB = (K + BK - 1) // BK

NEG = -3.0e38
BIGI = 2**30


def _topk_body(qn_ref, kb_ref, idx_ref, cand_d, cand_i):
    b = pl.program_id(0)
    kb = kb_ref[...]
    qn = qn_ref[...]
    norm = jnp.sqrt(jnp.sum(kb * kb, axis=1, keepdims=True))
    knb = kb / jnp.maximum(norm, 1e-12)
    sim = lax.dot_general(qn, knb, (((1,), (1,)), ((), ())),
                          preferred_element_type=jnp.float32)
    d = 1.0 - sim

    gidx = b * BK + lax.broadcasted_iota(jnp.int32, (Q, BK), 1)
    d = jnp.where(gidx < K, d, NEG)

    for t in range(NN):
        m = jnp.max(d, axis=1, keepdims=True)
        sel = jnp.min(jnp.where(d == m, gidx, BIGI), axis=1,
                      keepdims=True)
        cand_d[:, NN + t : NN + t + 1] = m
        cand_i[:, NN + t : NN + t + 1] = sel
        d = jnp.where(gidx == sel, NEG, d)

    @pl.when(b == 0)
    def _init():
        cand_d[:, 0:NN] = jnp.full((Q, NN), NEG, jnp.float32)
        cand_i[:, 0:NN] = jnp.full((Q, NN), BIGI, jnp.int32)

    cd = cand_d[...]
    ci = cand_i[...]
    for t in range(NN):
        m = jnp.max(cd, axis=1, keepdims=True)
        sel = jnp.min(jnp.where(cd == m, ci, BIGI), axis=1, keepdims=True)
        cand_d[:, t : t + 1] = m
        cand_i[:, t : t + 1] = sel
        cd = jnp.where(ci == sel, NEG, cd)

    @pl.when(b == NB - 1)
    def _emit():
        idx_ref[...] = cand_i[:, 0:NN]


def _topk_indices(qn, bank_features):
    return pl.pallas_call(
        _topk_body,
        grid=(NB,),
        in_specs=[
            pl.BlockSpec((Q, D), lambda b: (0, 0)),
            pl.BlockSpec((BK, D), lambda b: (b, 0)),
        ],
        out_specs=pl.BlockSpec((Q, NN), lambda b: (0, 0)),
        out_shape=jax.ShapeDtypeStruct((Q, NN), jnp.int32),
        scratch_shapes=[
            pltpu.VMEM((Q, 2 * NN), jnp.float32),
            pltpu.VMEM((Q, 2 * NN), jnp.int32),
        ],
    )(qn, bank_features)


_NC, _NS, _L = 2, 16, 16
_NW = _NC * _NS
_QPW = Q // _NW


def _feat_body(idx_hbm, feats_hbm, feats_out, idx_v, frow_v, fout_v, semf):
    wid = lax.axis_index("s") * _NC + lax.axis_index("c")
    base = wid * _QPW
    pltpu.sync_copy(idx_hbm.at[pl.ds(base, _QPW)], idx_v)

    inv = jnp.float32(1.0 / NN)

    def one_query(ql, _):
        idxrow = idx_v.at[ql]
        pltpu.async_copy(feats_hbm.at[idxrow], frow_v, semf).wait()
        for off in range(0, D, _L):
            s = jnp.zeros((_L,), jnp.float32)
            for r in range(NN):
                s = s + frow_v[r, pl.ds(off, _L)]
            fout_v[pl.ds(off, _L)] = s * inv
        pltpu.sync_copy(fout_v, feats_out.at[base + ql])
        return ()

    lax.fori_loop(0, _QPW, one_query, (), unroll=False)


@functools.lru_cache(maxsize=1)
def _feat_mean_kernel():
    return pl.kernel(
        _feat_body,
        out_type=jax.ShapeDtypeStruct((Q, D), jnp.float32),
        mesh=plsc.VectorSubcoreMesh(core_axis_name="c",
                                    subcore_axis_name="s"),
        compiler_params=pltpu.CompilerParams(needs_layout_passes=False),
        scratch_types=[
            pltpu.VMEM((_QPW, NN), jnp.int32),
            pltpu.VMEM((NN, D), jnp.float32),
            pltpu.VMEM((D,), jnp.float32),
            pltpu.SemaphoreType.DMA,
        ],
    )


GQ = 8
NG = Q // GQ


def _probs_body(idx_ref, *refs):
    bands = refs[: GQ * NN]
    probs_ref = refs[GQ * NN]
    lbl_ref = refs[GQ * NN + 1]
    g = pl.program_id(0)
    inv = jnp.float32(1.0 / NN)
    cidx = lax.broadcasted_iota(jnp.int32, (1, C), 1)
    siota = lax.broadcasted_iota(jnp.int32, (8, 1), 0)
    for j in range(GQ):
        acc = jnp.zeros((8, C), jnp.float32)
        for n in range(NN):
            sub = idx_ref[g * GQ + j, n] % 8
            acc = acc + jnp.where(siota == sub, bands[j * NN + n][...], 0.0)
        p = jnp.sum(acc, axis=0, keepdims=True) * inv
        probs_ref[j : j + 1, :] = p
        m = jnp.max(p, axis=1, keepdims=True)
        lbl_ref[j : j + 1, :] = jnp.min(
            jnp.where(p == m, cidx, BIGI), axis=1, keepdims=True)


def _probs_labels(idxs, bank_probs):
    row_specs = [
        pl.BlockSpec(
            (8, C),
            functools.partial(
                lambda j, n, g, idx_ref:
                (idx_ref[g * GQ + j, n] // 8, 0), j, n))
        for j in range(GQ)
        for n in range(NN)
    ]
    grid_spec = pltpu.PrefetchScalarGridSpec(
        num_scalar_prefetch=1,
        grid=(NG,),
        in_specs=[row_specs[i] for i in range(GQ * NN)],
        out_specs=[
            pl.BlockSpec((GQ, C), lambda g, idx_ref: (g, 0)),
            pl.BlockSpec((GQ, 1), lambda g, idx_ref: (g, 0)),
        ],
    )
    probs, lbl = pl.pallas_call(
        _probs_body,
        grid_spec=grid_spec,
        out_shape=[
            jax.ShapeDtypeStruct((Q, C), jnp.float32),
            jax.ShapeDtypeStruct((Q, 1), jnp.int32),
        ],
    )(idxs, *([bank_probs] * (GQ * NN)))
    return probs, lbl


def kernel(queries, bank_features, bank_probs):
    qnorm = jnp.linalg.norm(queries, axis=1, keepdims=True)
    qn = queries / jnp.maximum(qnorm, 1e-12)
    idxs = _topk_indices(qn, bank_features)
    grads = _feat_mean_kernel()(idxs, bank_features)
    pred_probs, lbl = _probs_labels(idxs, bank_probs)
    return (lbl.reshape(Q), pred_probs, grads)


# speedup vs baseline: 1.3495x; 1.0058x over previous
import functools

import jax
import jax.numpy as jnp
from jax import lax
from jax.experimental import pallas as pl
from jax.experimental.pallas import tpu as pltpu
from jax.experimental.pallas import tpu_sc as plsc

Q = 1024
K = 100000
D = 128
C = 1000
NN = 8

BK = 2048
NB = (K + BK - 1) // BK

NEG = -3.0e38
BIGI = 2**30


def _topk_body(qn_ref, kb_ref, idx_ref, cand_d, cand_i):
    b = pl.program_id(0)
    kb = kb_ref[...]
    qn = qn_ref[...]
    norm = jnp.sqrt(jnp.sum(kb * kb, axis=1, keepdims=True))
    knb = kb / jnp.maximum(norm, 1e-12)
    sim = lax.dot_general(qn, knb, (((1,), (1,)), ((), ())),
                          preferred_element_type=jnp.float32)
    d = 1.0 - sim

    gidx = b * BK + lax.broadcasted_iota(jnp.int32, (Q, BK), 1)
    d = jnp.where(gidx < K, d, NEG)

    for t in range(NN):
        m = jnp.max(d, axis=1, keepdims=True)
        sel = jnp.min(jnp.where(d == m, gidx, BIGI), axis=1,
                      keepdims=True)
        cand_d[:, NN + t : NN + t + 1] = m
        cand_i[:, NN + t : NN + t + 1] = sel
        d = jnp.where(gidx == sel, NEG, d)

    @pl.when(b == 0)
    def _init():
        cand_d[:, 0:NN] = jnp.full((Q, NN), NEG, jnp.float32)
        cand_i[:, 0:NN] = jnp.full((Q, NN), BIGI, jnp.int32)

    cd = cand_d[...]
    ci = cand_i[...]
    for t in range(NN):
        m = jnp.max(cd, axis=1, keepdims=True)
        sel = jnp.min(jnp.where(cd == m, ci, BIGI), axis=1, keepdims=True)
        cand_d[:, t : t + 1] = m
        cand_i[:, t : t + 1] = sel
        cd = jnp.where(ci == sel, NEG, cd)

    @pl.when(b == NB - 1)
    def _emit():
        idx_ref[...] = cand_i[:, 0:NN]


def _topk_indices(qn, bank_features):
    return pl.pallas_call(
        _topk_body,
        grid=(NB,),
        in_specs=[
            pl.BlockSpec((Q, D), lambda b: (0, 0)),
            pl.BlockSpec((BK, D), lambda b: (b, 0)),
        ],
        out_specs=pl.BlockSpec((Q, NN), lambda b: (0, 0)),
        out_shape=jax.ShapeDtypeStruct((Q, NN), jnp.int32),
        scratch_shapes=[
            pltpu.VMEM((Q, 2 * NN), jnp.float32),
            pltpu.VMEM((Q, 2 * NN), jnp.int32),
        ],
    )(qn, bank_features)


_NC, _NS, _L = 2, 16, 16
_NW = _NC * _NS
_QPW = Q // _NW


def _feat_body(idx_hbm, feats_hbm, feats_out, idx_v, frow_v, fout_v, semf):
    wid = lax.axis_index("s") * _NC + lax.axis_index("c")
    base = wid * _QPW
    pltpu.sync_copy(idx_hbm.at[pl.ds(base, _QPW)], idx_v)

    inv = jnp.float32(1.0 / NN)

    def one_query(ql, _):
        idxrow = idx_v.at[ql]
        pltpu.async_copy(feats_hbm.at[idxrow], frow_v, semf).wait()
        for off in range(0, D, _L):
            s = jnp.zeros((_L,), jnp.float32)
            for r in range(NN):
                s = s + frow_v[r, pl.ds(off, _L)]
            fout_v[pl.ds(off, _L)] = s * inv
        pltpu.sync_copy(fout_v, feats_out.at[base + ql])
        return ()

    lax.fori_loop(0, _QPW, one_query, (), unroll=False)


@functools.lru_cache(maxsize=1)
def _feat_mean_kernel():
    return pl.kernel(
        _feat_body,
        out_type=jax.ShapeDtypeStruct((Q, D), jnp.float32),
        mesh=plsc.VectorSubcoreMesh(core_axis_name="c",
                                    subcore_axis_name="s"),
        compiler_params=pltpu.CompilerParams(needs_layout_passes=False),
        scratch_types=[
            pltpu.VMEM((_QPW, NN), jnp.int32),
            pltpu.VMEM((NN, D), jnp.float32),
            pltpu.VMEM((D,), jnp.float32),
            pltpu.SemaphoreType.DMA,
        ],
    )


GQ = 16
NG = Q // GQ


def _probs_body(idx_ref, *refs):
    bands = refs[: GQ * NN]
    probs_ref = refs[GQ * NN]
    lbl_ref = refs[GQ * NN + 1]
    g = pl.program_id(0)
    inv = jnp.float32(1.0 / NN)
    cidx = lax.broadcasted_iota(jnp.int32, (1, C), 1)
    siota = lax.broadcasted_iota(jnp.int32, (8, 1), 0)
    for j in range(GQ):
        acc = jnp.zeros((8, C), jnp.float32)
        for n in range(NN):
            sub = idx_ref[g * GQ + j, n] % 8
            acc = acc + jnp.where(siota == sub, bands[j * NN + n][...], 0.0)
        p = jnp.sum(acc, axis=0, keepdims=True) * inv
        probs_ref[j : j + 1, :] = p
        m = jnp.max(p, axis=1, keepdims=True)
        lbl_ref[j : j + 1, :] = jnp.min(
            jnp.where(p == m, cidx, BIGI), axis=1, keepdims=True)


def _probs_labels(idxs, bank_probs):
    row_specs = [
        pl.BlockSpec(
            (8, C),
            functools.partial(
                lambda j, n, g, idx_ref:
                (idx_ref[g * GQ + j, n] // 8, 0), j, n))
        for j in range(GQ)
        for n in range(NN)
    ]
    grid_spec = pltpu.PrefetchScalarGridSpec(
        num_scalar_prefetch=1,
        grid=(NG,),
        in_specs=[row_specs[i] for i in range(GQ * NN)],
        out_specs=[
            pl.BlockSpec((GQ, C), lambda g, idx_ref: (g, 0)),
            pl.BlockSpec((GQ, 1), lambda g, idx_ref: (g, 0)),
        ],
    )
    probs, lbl = pl.pallas_call(
        _probs_body,
        grid_spec=grid_spec,
        out_shape=[
            jax.ShapeDtypeStruct((Q, C), jnp.float32),
            jax.ShapeDtypeStruct((Q, 1), jnp.int32),
        ],
    )(idxs, *([bank_probs] * (GQ * NN)))
    return probs, lbl


def kernel(queries, bank_features, bank_probs):
    qnorm = jnp.linalg.norm(queries, axis=1, keepdims=True)
    qn = queries / jnp.maximum(qnorm, 1e-12)
    idxs = _topk_indices(qn, bank_features)
    grads = _feat_mean_kernel()(idxs, bank_features)
    pred_probs, lbl = _probs_labels(idxs, bank_probs)
    return (lbl.reshape(Q), pred_probs, grads)
